# Initial kernel scaffold; baseline (speedup 1.0000x reference)
import jax
import jax.numpy as jnp
from jax.experimental import pallas as pl


def kernel(feat_map, ins_ids, ins_feats):
    raise NotImplementedError("write your pallas kernel here")


# speedup vs baseline: 1.4198x; 1.4198x over previous
import functools

import jax
import jax.numpy as jnp
from jax import lax
from jax.experimental import pallas as pl
from jax.experimental.pallas import tpu as pltpu
from jax.experimental.pallas import tpu_sc as plsc

B, C, FH, FW = 8, 256, 128, 128
HW = FH * FW
N = 4096
L = 16
NUM_TILES = 32
TILES_PER_BATCH = NUM_TILES // B
C_PER_TILE = C // TILES_PER_BATCH
NGROUPS = N // L
NBUF = 4


def _transpose_body(x_ref, o_ref):
    o_ref[0] = jnp.swapaxes(x_ref[0], 0, 1)


def _transpose_feats(ins_feats):
    nt = 8
    nb = N // nt
    return pl.pallas_call(
        _transpose_body,
        grid=(B, nt),
        in_specs=[pl.BlockSpec((1, nb, C), lambda b, i: (b, i, 0))],
        out_specs=pl.BlockSpec((1, C, nb), lambda b, i: (b, 0, i)),
        out_shape=jax.ShapeDtypeStruct((B, C, N), jnp.float32),
    )(ins_feats)


def _sc_body(fm_hbm, ids_hbm, vals_hbm, out_hbm,
             ids_v, claim_v, win_v,
             vals_v0, vals_v1, vals_v2, vals_v3,
             row_v0, row_v1, row_v2, row_v3, sem_in, sem_out):
    vals_bufs = [vals_v0, vals_v1, vals_v2, vals_v3]
    row_bufs = [row_v0, row_v1, row_v2, row_v3]
    wid = lax.axis_index("s") * 2 + lax.axis_index("c")
    b = wid // TILES_PER_BATCH
    c0 = (wid % TILES_PER_BATCH) * C_PER_TILE

    iota = lax.iota(jnp.int32, L)

    pltpu.sync_copy(ids_hbm.at[b], ids_v)

    def claim_body(g, _):
        k = ids_v[pl.ds(g * L, L)]
        nvec = g * L + iota
        _, last = plsc.scan_count(k)
        plsc.store_scatter(claim_v, [k], nvec, mask=last)
        return 0

    lax.fori_loop(0, NGROUPS, claim_body, 0)

    def win_body(g, _):
        k = ids_v[pl.ds(g * L, L)]
        nvec = g * L + iota
        w = plsc.load_gather(claim_v, [k])
        win_v[pl.ds(g * L, L)] = jnp.where(w == nvec, 1, 0)
        return 0

    lax.fori_loop(0, NGROUPS, win_body, 0)

    def start_in(r, s):
        pltpu.async_copy(fm_hbm.at[b, c0 + r], row_bufs[s], sem_in.at[s])
        pltpu.async_copy(vals_hbm.at[b, c0 + r], vals_bufs[s], sem_in.at[s])

    def wait_in(s):
        pltpu.make_async_copy(fm_hbm.at[b, c0], row_bufs[s],
                              sem_in.at[s]).wait()
        pltpu.make_async_copy(vals_hbm.at[b, c0], vals_bufs[s],
                              sem_in.at[s]).wait()

    def start_out(r, s):
        pltpu.async_copy(row_bufs[s], out_hbm.at[b, c0 + r], sem_out.at[s])

    def wait_out(s):
        pltpu.make_async_copy(row_bufs[s], out_hbm.at[b, c0],
                              sem_out.at[s]).wait()

    start_in(0, 0)
    start_in(1, 1)

    def chunk_body(chunk):
        for s in range(NBUF):
            r = chunk * NBUF + s
            wait_in(s)

            def scat_body(g, _):
                k = ids_v[pl.ds(g * L, L)]
                v = vals_bufs[s][pl.ds(g * L, L)]
                m = win_v[pl.ds(g * L, L)] == 1
                plsc.store_scatter(row_bufs[s], [k], v, mask=m)
                return 0

            lax.fori_loop(0, NGROUPS, scat_body, 0)
            start_out(r, s)

            s2 = (s + 2) % NBUF

            @pl.when(jnp.logical_and(r >= 2, r + 2 < C_PER_TILE))
            def _():
                wait_out(s2)

            @pl.when(r + 2 < C_PER_TILE)
            def _():
                start_in(r + 2, s2)

    pl.loop(0, C_PER_TILE // NBUF)(chunk_body)

    for s in range(NBUF):
        wait_out(s)


@functools.partial(
    pl.kernel,
    out_type=jax.ShapeDtypeStruct((B, C, HW), jnp.float32),
    mesh=plsc.VectorSubcoreMesh(core_axis_name="c", subcore_axis_name="s"),
    compiler_params=pltpu.CompilerParams(needs_layout_passes=False),
    scratch_types=[
        pltpu.VMEM((N,), jnp.int32),
        pltpu.VMEM((HW,), jnp.int32),
        pltpu.VMEM((N,), jnp.int32),
        pltpu.VMEM((N,), jnp.float32),
        pltpu.VMEM((N,), jnp.float32),
        pltpu.VMEM((N,), jnp.float32),
        pltpu.VMEM((N,), jnp.float32),
        pltpu.VMEM((HW,), jnp.float32),
        pltpu.VMEM((HW,), jnp.float32),
        pltpu.VMEM((HW,), jnp.float32),
        pltpu.VMEM((HW,), jnp.float32),
        pltpu.SemaphoreType.DMA((NBUF,)),
        pltpu.SemaphoreType.DMA((NBUF,)),
    ],
)
def _sc_insert(fm_hbm, ids_hbm, vals_hbm, out_hbm,
               ids_v, claim_v, win_v,
               vals_v0, vals_v1, vals_v2, vals_v3,
               row_v0, row_v1, row_v2, row_v3, sem_in, sem_out):
    _sc_body(fm_hbm, ids_hbm, vals_hbm, out_hbm,
             ids_v, claim_v, win_v,
             vals_v0, vals_v1, vals_v2, vals_v3,
             row_v0, row_v1, row_v2, row_v3, sem_in, sem_out)


def kernel(feat_map, ins_ids, ins_feats):
    fm3 = feat_map.reshape(B, C, HW)
    ids = ins_ids.astype(jnp.int32)
    vals_t = _transpose_feats(ins_feats.astype(jnp.float32))
    out3 = _sc_insert(fm3, ids, vals_t)
    return out3.reshape(B, C, FH, FW)


# speedup vs baseline: 3.3772x; 2.3787x over previous
import functools

import jax
import jax.numpy as jnp
from jax import lax
from jax.experimental import pallas as pl
from jax.experimental.pallas import tpu as pltpu
from jax.experimental.pallas import tpu_sc as plsc

B, C, FH, FW = 8, 256, 128, 128
HW = FH * FW
N = 4096
NR = N // FW
L = 16
NUM_TILES = 32
TILES_PER_BATCH = NUM_TILES // B
C_PER_TILE = C // TILES_PER_BATCH
NGROUPS = N // L
NBUF = 4


def _transpose_body(x_ref, o_ref):
    o_ref[0] = jnp.swapaxes(x_ref[0], 0, 1).reshape(C * N // FW, FW)


def _transpose_feats(ins_feats):
    return pl.pallas_call(
        _transpose_body,
        grid=(B,),
        in_specs=[pl.BlockSpec((1, N, C), lambda b: (b, 0, 0))],
        out_specs=pl.BlockSpec((1, C * N // FW, FW), lambda b: (b, 0, 0)),
        out_shape=jax.ShapeDtypeStruct((B, C * N // FW, FW), jnp.float32),
    )(ins_feats)


def _sc_body(fm_hbm, ids_hbm, vals_hbm, out_hbm,
             ids_v, claim_v, win_v,
             vals_bufs, row_bufs, sem_in, sem_out):
    wid = lax.axis_index("s") * 2 + lax.axis_index("c")
    b = wid // TILES_PER_BATCH
    c0 = (wid % TILES_PER_BATCH) * C_PER_TILE

    iota = lax.iota(jnp.int32, L)

    pltpu.sync_copy(ids_hbm.at[b], ids_v)

    def claim_body(g, _):
        k = ids_v[g >> 3, pl.ds((g & 7) * L, L)]
        nvec = g * L + iota
        _, last = plsc.scan_count(k)
        plsc.store_scatter(claim_v, [k], nvec, mask=last)
        return 0

    lax.fori_loop(0, NGROUPS, claim_body, 0)

    def win_body(g, _):
        k = ids_v[g >> 3, pl.ds((g & 7) * L, L)]
        nvec = g * L + iota
        w = plsc.load_gather(claim_v, [k])
        win_v[pl.ds(g * L, L)] = jnp.where(w == nvec, 1, 0)
        return 0

    lax.fori_loop(0, NGROUPS, win_body, 0)

    def start_in(r, s):
        pltpu.async_copy(fm_hbm.at[b, c0 + r], row_bufs[s], sem_in.at[s])
        pltpu.async_copy(vals_hbm.at[b, pl.ds((c0 + r) * NR, NR)],
                         vals_bufs[s], sem_in.at[s])

    def wait_in(s):
        pltpu.make_async_copy(fm_hbm.at[b, c0], row_bufs[s],
                              sem_in.at[s]).wait()
        pltpu.make_async_copy(vals_hbm.at[b, pl.ds(0, NR)], vals_bufs[s],
                              sem_in.at[s]).wait()

    def start_out(r, s):
        pltpu.async_copy(row_bufs[s], out_hbm.at[b, c0 + r], sem_out.at[s])

    def wait_out(s):
        pltpu.make_async_copy(row_bufs[s], out_hbm.at[b, c0],
                              sem_out.at[s]).wait()

    start_in(0, 0)
    start_in(1, 1)

    def chunk_body(chunk):
        for s in range(NBUF):
            r = chunk * NBUF + s
            wait_in(s)

            def scat_body(g, _):
                k = ids_v[g >> 3, pl.ds((g & 7) * L, L)]
                v = vals_bufs[s][g >> 3, pl.ds((g & 7) * L, L)]
                m = win_v[pl.ds(g * L, L)] == 1
                plsc.store_scatter(row_bufs[s], [k >> 7, k & 127], v, mask=m)
                return 0

            lax.fori_loop(0, NGROUPS, scat_body, 0)
            start_out(r, s)

            s2 = (s + 2) % NBUF

            @pl.when(jnp.logical_and(r >= 2, r + 2 < C_PER_TILE))
            def _():
                wait_out(s2)

            @pl.when(r + 2 < C_PER_TILE)
            def _():
                start_in(r + 2, s2)

    pl.loop(0, C_PER_TILE // NBUF)(chunk_body)

    for s in range(NBUF):
        wait_out(s)


@functools.partial(
    pl.kernel,
    out_type=jax.ShapeDtypeStruct((B, C, FH, FW), jnp.float32),
    mesh=plsc.VectorSubcoreMesh(core_axis_name="c", subcore_axis_name="s"),
    compiler_params=pltpu.CompilerParams(needs_layout_passes=False),
    scratch_types=[
        pltpu.VMEM((NR, FW), jnp.int32),
        pltpu.VMEM((HW,), jnp.int32),
        pltpu.VMEM((N,), jnp.int32),
        pltpu.VMEM((NR, FW), jnp.float32),
        pltpu.VMEM((NR, FW), jnp.float32),
        pltpu.VMEM((NR, FW), jnp.float32),
        pltpu.VMEM((NR, FW), jnp.float32),
        pltpu.VMEM((FH, FW), jnp.float32),
        pltpu.VMEM((FH, FW), jnp.float32),
        pltpu.VMEM((FH, FW), jnp.float32),
        pltpu.VMEM((FH, FW), jnp.float32),
        pltpu.SemaphoreType.DMA((NBUF,)),
        pltpu.SemaphoreType.DMA((NBUF,)),
    ],
)
def _sc_insert(fm_hbm, ids_hbm, vals_hbm, out_hbm,
               ids_v, claim_v, win_v,
               vals_v0, vals_v1, vals_v2, vals_v3,
               row_v0, row_v1, row_v2, row_v3, sem_in, sem_out):
    _sc_body(fm_hbm, ids_hbm, vals_hbm, out_hbm,
             ids_v, claim_v, win_v,
             [vals_v0, vals_v1, vals_v2, vals_v3],
             [row_v0, row_v1, row_v2, row_v3], sem_in, sem_out)


def kernel(feat_map, ins_ids, ins_feats):
    ids3 = ins_ids.astype(jnp.int32).reshape(B, NR, FW)
    vals3 = _transpose_feats(ins_feats.astype(jnp.float32))
    out4 = _sc_insert(feat_map, ids3, vals3)
    return out4


# speedup vs baseline: 4.0230x; 1.1912x over previous
import functools

import jax
import jax.numpy as jnp
from jax import lax
from jax.experimental import pallas as pl
from jax.experimental.pallas import tpu as pltpu
from jax.experimental.pallas import tpu_sc as plsc

B, C, FH, FW = 8, 256, 128, 128
HW = FH * FW
N = 4096
NR = N // FW
L = 16
NUM_TILES = 32
TILES_PER_BATCH = NUM_TILES // B
C_PER_TILE = C // TILES_PER_BATCH
NGROUPS = N // L
NBUF = 4
PAD = 8


def _transpose_body(x_ref, o_ref):
    o_ref[0] = jnp.swapaxes(x_ref[0], 0, 1).reshape(C // 2 * NR, FW)


def _transpose_feats(ins_feats):
    return pl.pallas_call(
        _transpose_body,
        grid=(B, 2),
        in_specs=[pl.BlockSpec((1, N, C // 2), lambda b, j: (b, 0, j))],
        out_specs=pl.BlockSpec((1, C // 2 * NR, FW), lambda b, j: (b, j, 0)),
        out_shape=jax.ShapeDtypeStruct((B, C * NR, FW), jnp.float32),
    )(ins_feats)


def _sc_body(fm_hbm, ids_hbm, vals_hbm, out_hbm,
             ids_v, claim_v, hh_v, ww_v,
             vals_bufs, row_bufs, sem_in, sem_out):
    wid = lax.axis_index("s") * 2 + lax.axis_index("c")
    b = wid // TILES_PER_BATCH
    c0 = (wid % TILES_PER_BATCH) * C_PER_TILE

    iota = lax.iota(jnp.int32, L)

    pltpu.sync_copy(ids_hbm.at[b], ids_v)

    def claim_body(g, _):
        k = ids_v[g >> 3, pl.ds((g & 7) * L, L)]
        nvec = g * L + iota
        _, last = plsc.scan_count(k)
        plsc.store_scatter(claim_v, [k], nvec, mask=last)
        return 0

    lax.fori_loop(0, NGROUPS, claim_body, 0)

    def win_body(g):
        k = ids_v[g >> 3, pl.ds((g & 7) * L, L)]
        nvec = g * L + iota
        w = plsc.load_gather(claim_v, [k])
        hh_v[pl.ds(g * L, L)] = jnp.where(w == nvec, k >> 7, FH)
        ww_v[pl.ds(g * L, L)] = k & 127

    plsc.parallel_loop(0, NGROUPS, unroll=2)(win_body)

    def start_in(r, s):
        pltpu.async_copy(fm_hbm.at[b, c0 + r],
                         row_bufs[s].at[pl.ds(0, FH)], sem_in.at[s])
        pltpu.async_copy(vals_hbm.at[b, pl.ds((c0 + r) * NR, NR)],
                         vals_bufs[s], sem_in.at[s])

    def wait_in(s):
        pltpu.make_async_copy(fm_hbm.at[b, c0],
                              row_bufs[s].at[pl.ds(0, FH)],
                              sem_in.at[s]).wait()
        pltpu.make_async_copy(vals_hbm.at[b, pl.ds(0, NR)], vals_bufs[s],
                              sem_in.at[s]).wait()

    def start_out(r, s):
        pltpu.async_copy(row_bufs[s].at[pl.ds(0, FH)],
                         out_hbm.at[b, c0 + r], sem_out.at[s])

    def wait_out(s):
        pltpu.make_async_copy(row_bufs[s].at[pl.ds(0, FH)],
                              out_hbm.at[b, c0], sem_out.at[s]).wait()

    start_in(0, 0)
    start_in(1, 1)

    def chunk_body(chunk):
        for s in range(NBUF):
            r = chunk * NBUF + s
            wait_in(s)

            def scat_body(g):
                h = hh_v[pl.ds(g * L, L)]
                w = ww_v[pl.ds(g * L, L)]
                v = vals_bufs[s][g >> 3, pl.ds((g & 7) * L, L)]
                plsc.store_scatter(row_bufs[s], [h, w], v)

            plsc.parallel_loop(0, NGROUPS, unroll=4)(scat_body)
            start_out(r, s)

            s2 = (s + 2) % NBUF

            @pl.when(jnp.logical_and(r >= 2, r + 2 < C_PER_TILE))
            def _():
                wait_out(s2)

            @pl.when(r + 2 < C_PER_TILE)
            def _():
                start_in(r + 2, s2)

    pl.loop(0, C_PER_TILE // NBUF)(chunk_body)

    for s in range(NBUF):
        wait_out(s)


@functools.partial(
    pl.kernel,
    out_type=jax.ShapeDtypeStruct((B, C, FH, FW), jnp.float32),
    mesh=plsc.VectorSubcoreMesh(core_axis_name="c", subcore_axis_name="s"),
    compiler_params=pltpu.CompilerParams(needs_layout_passes=False),
    scratch_types=[
        pltpu.VMEM((NR, FW), jnp.int32),
        pltpu.VMEM((HW,), jnp.int32),
        pltpu.VMEM((N,), jnp.int32),
        pltpu.VMEM((N,), jnp.int32),
        pltpu.VMEM((NR, FW), jnp.float32),
        pltpu.VMEM((NR, FW), jnp.float32),
        pltpu.VMEM((NR, FW), jnp.float32),
        pltpu.VMEM((NR, FW), jnp.float32),
        pltpu.VMEM((FH + PAD, FW), jnp.float32),
        pltpu.VMEM((FH + PAD, FW), jnp.float32),
        pltpu.VMEM((FH + PAD, FW), jnp.float32),
        pltpu.VMEM((FH + PAD, FW), jnp.float32),
        pltpu.SemaphoreType.DMA((NBUF,)),
        pltpu.SemaphoreType.DMA((NBUF,)),
    ],
)
def _sc_insert(fm_hbm, ids_hbm, vals_hbm, out_hbm,
               ids_v, claim_v, hh_v, ww_v,
               vals_v0, vals_v1, vals_v2, vals_v3,
               row_v0, row_v1, row_v2, row_v3, sem_in, sem_out):
    _sc_body(fm_hbm, ids_hbm, vals_hbm, out_hbm,
             ids_v, claim_v, hh_v, ww_v,
             [vals_v0, vals_v1, vals_v2, vals_v3],
             [row_v0, row_v1, row_v2, row_v3], sem_in, sem_out)


def kernel(feat_map, ins_ids, ins_feats):
    ids3 = ins_ids.astype(jnp.int32).reshape(B, NR, FW)
    vals3 = _transpose_feats(ins_feats.astype(jnp.float32))
    out4 = _sc_insert(feat_map, ids3, vals3)
    return out4


# speedup vs baseline: 4.0776x; 1.0136x over previous
import functools

import jax
import jax.numpy as jnp
from jax import lax
from jax.experimental import pallas as pl
from jax.experimental.pallas import tpu as pltpu
from jax.experimental.pallas import tpu_sc as plsc

B, C, FH, FW = 8, 256, 128, 128
HW = FH * FW
N = 4096
NR = N // FW
L = 16
NUM_TILES = 32
TILES_PER_BATCH = NUM_TILES // B
C_PER_TILE = C // TILES_PER_BATCH
NGROUPS = N // L
NBUF = 4
PAD = 8


def _transpose_body(x_ref, o_ref):
    o_ref[0] = jnp.swapaxes(x_ref[0], 0, 1).reshape(C // 2 * NR, FW)


def _transpose_feats(ins_feats):
    return pl.pallas_call(
        _transpose_body,
        grid=(B, 2),
        in_specs=[pl.BlockSpec((1, N, C // 2), lambda b, j: (b, 0, j))],
        out_specs=pl.BlockSpec((1, C // 2 * NR, FW), lambda b, j: (b, j, 0)),
        out_shape=jax.ShapeDtypeStruct((B, C * NR, FW), jnp.float32),
    )(ins_feats)


def _sc_body(fm_hbm, ids_hbm, vals_hbm, out_hbm,
             ids_v, claim_v, hw_v,
             vals_bufs, row_bufs, sem_in, sem_out):
    wid = lax.axis_index("s") * 2 + lax.axis_index("c")
    b = wid // TILES_PER_BATCH
    c0 = (wid % TILES_PER_BATCH) * C_PER_TILE

    iota = lax.iota(jnp.int32, L)

    def start_in(r, s):
        pltpu.async_copy(fm_hbm.at[b, c0 + r],
                         row_bufs[s].at[pl.ds(0, FH)], sem_in.at[s])
        pltpu.async_copy(vals_hbm.at[b, pl.ds((c0 + r) * NR, NR)],
                         vals_bufs[s], sem_in.at[s])

    start_in(0, 0)
    start_in(1, 1)

    pltpu.sync_copy(ids_hbm.at[b], ids_v)

    def claim_body(g, _):
        k = ids_v[g >> 3, pl.ds((g & 7) * L, L)]
        nvec = g * L + iota
        _, last = plsc.scan_count(k)
        plsc.store_scatter(claim_v, [k], nvec, mask=last)
        return 0

    lax.fori_loop(0, NGROUPS, claim_body, 0)

    def win_body(g):
        k = ids_v[g >> 3, pl.ds((g & 7) * L, L)]
        nvec = g * L + iota
        w = plsc.load_gather(claim_v, [k])
        hw_v[pl.ds(g * L, L)] = jnp.where(w == nvec, k, (k & 127) | HW)

    plsc.parallel_loop(0, NGROUPS, unroll=2)(win_body)

    def wait_in(s):
        pltpu.make_async_copy(fm_hbm.at[b, c0],
                              row_bufs[s].at[pl.ds(0, FH)],
                              sem_in.at[s]).wait()
        pltpu.make_async_copy(vals_hbm.at[b, pl.ds(0, NR)], vals_bufs[s],
                              sem_in.at[s]).wait()

    def start_out(r, s):
        pltpu.async_copy(row_bufs[s].at[pl.ds(0, FH)],
                         out_hbm.at[b, c0 + r], sem_out.at[s])

    def wait_out(s):
        pltpu.make_async_copy(row_bufs[s].at[pl.ds(0, FH)],
                              out_hbm.at[b, c0], sem_out.at[s]).wait()

    def chunk_body(chunk):
        for s in range(NBUF):
            r = chunk * NBUF + s
            wait_in(s)

            def scat_body(g):
                hw = hw_v[pl.ds(g * L, L)]
                v = vals_bufs[s][g >> 3, pl.ds((g & 7) * L, L)]
                plsc.store_scatter(row_bufs[s], [hw >> 7, hw & 127], v)

            plsc.parallel_loop(0, NGROUPS, unroll=4)(scat_body)
            start_out(r, s)

            s2 = (s + 2) % NBUF

            @pl.when(jnp.logical_and(r >= 2, r + 2 < C_PER_TILE))
            def _():
                wait_out(s2)

            @pl.when(r + 2 < C_PER_TILE)
            def _():
                start_in(r + 2, s2)

    pl.loop(0, C_PER_TILE // NBUF)(chunk_body)

    for s in range(NBUF):
        wait_out(s)


@functools.partial(
    pl.kernel,
    out_type=jax.ShapeDtypeStruct((B, C, FH, FW), jnp.float32),
    mesh=plsc.VectorSubcoreMesh(core_axis_name="c", subcore_axis_name="s"),
    compiler_params=pltpu.CompilerParams(needs_layout_passes=False),
    scratch_types=[
        pltpu.VMEM((NR, FW), jnp.int32),
        pltpu.VMEM((HW,), jnp.int32),
        pltpu.VMEM((N,), jnp.int32),
        pltpu.VMEM((NR, FW), jnp.float32),
        pltpu.VMEM((NR, FW), jnp.float32),
        pltpu.VMEM((NR, FW), jnp.float32),
        pltpu.VMEM((NR, FW), jnp.float32),
        pltpu.VMEM((FH + PAD, FW), jnp.float32),
        pltpu.VMEM((FH + PAD, FW), jnp.float32),
        pltpu.VMEM((FH + PAD, FW), jnp.float32),
        pltpu.VMEM((FH + PAD, FW), jnp.float32),
        pltpu.SemaphoreType.DMA((NBUF,)),
        pltpu.SemaphoreType.DMA((NBUF,)),
    ],
)
def _sc_insert(fm_hbm, ids_hbm, vals_hbm, out_hbm,
               ids_v, claim_v, hw_v,
               vals_v0, vals_v1, vals_v2, vals_v3,
               row_v0, row_v1, row_v2, row_v3, sem_in, sem_out):
    _sc_body(fm_hbm, ids_hbm, vals_hbm, out_hbm,
             ids_v, claim_v, hw_v,
             [vals_v0, vals_v1, vals_v2, vals_v3],
             [row_v0, row_v1, row_v2, row_v3], sem_in, sem_out)


def kernel(feat_map, ins_ids, ins_feats):
    ids3 = ins_ids.astype(jnp.int32).reshape(B, NR, FW)
    vals3 = _transpose_feats(ins_feats.astype(jnp.float32))
    out4 = _sc_insert(feat_map, ids3, vals3)
    return out4


# speedup vs baseline: 4.1274x; 1.0122x over previous
import functools

import jax
import jax.numpy as jnp
from jax import lax
from jax.experimental import pallas as pl
from jax.experimental.pallas import tpu as pltpu
from jax.experimental.pallas import tpu_sc as plsc

B, C, FH, FW = 8, 256, 128, 128
HW = FH * FW
N = 4096
NR = N // FW
L = 16
NUM_TILES = 32
TILES_PER_BATCH = NUM_TILES // B
C_PER_TILE = C // TILES_PER_BATCH
NGROUPS = N // L
NBUF = 4
PAD = 8


def _transpose_body(x_ref, o_ref):
    o_ref[0] = jnp.swapaxes(x_ref[0], 0, 1).reshape(C // 2 * NR, FW)


def _transpose_feats(ins_feats):
    return pl.pallas_call(
        _transpose_body,
        grid=(B, 2),
        in_specs=[pl.BlockSpec((1, N, C // 2), lambda b, j: (b, 0, j))],
        out_specs=pl.BlockSpec((1, C // 2 * NR, FW), lambda b, j: (b, j, 0)),
        out_shape=jax.ShapeDtypeStruct((B, C * NR, FW), jnp.float32),
    )(ins_feats)


def _sc_body(fm_hbm, ids_hbm, vals_hbm, out_hbm,
             ids_v, claim_v, hw_v,
             vals_bufs, row_bufs, sem_in, sem_out):
    wid = lax.axis_index("s") * 2 + lax.axis_index("c")
    b = wid // TILES_PER_BATCH
    c0 = (wid % TILES_PER_BATCH) * C_PER_TILE

    iota = lax.iota(jnp.int32, L)

    def start_in(r, s):
        pltpu.async_copy(fm_hbm.at[b, c0 + r],
                         row_bufs[s].at[pl.ds(0, FH)], sem_in.at[s])
        pltpu.async_copy(vals_hbm.at[b, pl.ds((c0 + r) * NR, NR)],
                         vals_bufs[s], sem_in.at[s])

    start_in(0, 0)
    start_in(1, 1)

    pltpu.sync_copy(ids_hbm.at[b], ids_v)

    def claim_body(g, _):
        k = ids_v[g >> 3, pl.ds((g & 7) * L, L)]
        nvec = g * L + iota
        _, last = plsc.scan_count(k)
        plsc.store_scatter(claim_v, [k], nvec, mask=last)
        return 0

    lax.fori_loop(0, NGROUPS, claim_body, 0)

    def win_body(g):
        k = ids_v[g >> 3, pl.ds((g & 7) * L, L)]
        nvec = g * L + iota
        w = plsc.load_gather(claim_v, [k])
        hw_v[pl.ds(g * L, L)] = jnp.where(w == nvec, k, (k & 127) | HW)

    plsc.parallel_loop(0, NGROUPS, unroll=2)(win_body)

    def wait_in(s):
        pltpu.make_async_copy(fm_hbm.at[b, c0],
                              row_bufs[s].at[pl.ds(0, FH)],
                              sem_in.at[s]).wait()
        pltpu.make_async_copy(vals_hbm.at[b, pl.ds(0, NR)], vals_bufs[s],
                              sem_in.at[s]).wait()

    def start_out(r, s):
        pltpu.async_copy(row_bufs[s].at[pl.ds(0, FH)],
                         out_hbm.at[b, c0 + r], sem_out.at[s])

    def wait_out(s):
        pltpu.make_async_copy(row_bufs[s].at[pl.ds(0, FH)],
                              out_hbm.at[b, c0], sem_out.at[s]).wait()

    def chunk_body(chunk):
        for s in range(NBUF):
            r = chunk * NBUF + s
            wait_in(s)
            s2 = (s + 2) % NBUF

            @pl.when(jnp.logical_and(r >= 2, r + 2 < C_PER_TILE))
            def _():
                wait_out(s2)

            @pl.when(r + 2 < C_PER_TILE)
            def _():
                start_in(r + 2, s2)

            def scat_body(g):
                hw = hw_v[pl.ds(g * L, L)]
                v = vals_bufs[s][g >> 3, pl.ds((g & 7) * L, L)]
                plsc.store_scatter(row_bufs[s], [hw >> 7, hw & 127], v)

            plsc.parallel_loop(0, NGROUPS, unroll=4)(scat_body)
            start_out(r, s)

    pl.loop(0, C_PER_TILE // NBUF)(chunk_body)

    for s in range(NBUF):
        wait_out(s)


@functools.partial(
    pl.kernel,
    out_type=jax.ShapeDtypeStruct((B, C, FH, FW), jnp.float32),
    mesh=plsc.VectorSubcoreMesh(core_axis_name="c", subcore_axis_name="s"),
    compiler_params=pltpu.CompilerParams(needs_layout_passes=False),
    scratch_types=[
        pltpu.VMEM((NR, FW), jnp.int32),
        pltpu.VMEM((HW,), jnp.int32),
        pltpu.VMEM((N,), jnp.int32),
        pltpu.VMEM((NR, FW), jnp.float32),
        pltpu.VMEM((NR, FW), jnp.float32),
        pltpu.VMEM((NR, FW), jnp.float32),
        pltpu.VMEM((NR, FW), jnp.float32),
        pltpu.VMEM((FH + PAD, FW), jnp.float32),
        pltpu.VMEM((FH + PAD, FW), jnp.float32),
        pltpu.VMEM((FH + PAD, FW), jnp.float32),
        pltpu.VMEM((FH + PAD, FW), jnp.float32),
        pltpu.SemaphoreType.DMA((NBUF,)),
        pltpu.SemaphoreType.DMA((NBUF,)),
    ],
)
def _sc_insert(fm_hbm, ids_hbm, vals_hbm, out_hbm,
               ids_v, claim_v, hw_v,
               vals_v0, vals_v1, vals_v2, vals_v3,
               row_v0, row_v1, row_v2, row_v3, sem_in, sem_out):
    _sc_body(fm_hbm, ids_hbm, vals_hbm, out_hbm,
             ids_v, claim_v, hw_v,
             [vals_v0, vals_v1, vals_v2, vals_v3],
             [row_v0, row_v1, row_v2, row_v3], sem_in, sem_out)


def kernel(feat_map, ins_ids, ins_feats):
    ids3 = ins_ids.astype(jnp.int32).reshape(B, NR, FW)
    vals3 = _transpose_feats(ins_feats.astype(jnp.float32))
    out4 = _sc_insert(feat_map, ids3, vals3)
    return out4


# speedup vs baseline: 4.1890x; 1.0149x over previous
import functools

import jax
import jax.numpy as jnp
from jax import lax
from jax.experimental import pallas as pl
from jax.experimental.pallas import tpu as pltpu
from jax.experimental.pallas import tpu_sc as plsc

B, C, FH, FW = 8, 256, 128, 128
HW = FH * FW
N = 4096
NR = N // FW
L = 16
NUM_TILES = 32
TILES_PER_BATCH = NUM_TILES // B
C_PER_TILE = C // TILES_PER_BATCH
NGROUPS = N // L
NBUF = 4
PAD = 8


def _transpose_body(x_ref, o_ref):
    o_ref[0] = jnp.swapaxes(x_ref[0], 0, 1).reshape(C // 2 * NR, FW)


def _transpose_feats(ins_feats):
    return pl.pallas_call(
        _transpose_body,
        grid=(B, 2),
        in_specs=[pl.BlockSpec((1, N, C // 2), lambda b, j: (b, 0, j))],
        out_specs=pl.BlockSpec((1, C // 2 * NR, FW), lambda b, j: (b, j, 0)),
        out_shape=jax.ShapeDtypeStruct((B, C * NR, FW), jnp.float32),
    )(ins_feats)


def _sc_body(fm_hbm, ids_hbm, vals_hbm, out_hbm,
             ids_v, claim_v, hw_v,
             vals_bufs, row_bufs, sem_in, sem_out):
    wid = lax.axis_index("s") * 2 + lax.axis_index("c")
    b = wid // TILES_PER_BATCH
    c0 = (wid % TILES_PER_BATCH) * C_PER_TILE

    iota = lax.iota(jnp.int32, L)

    def start_in(r, s):
        pltpu.async_copy(fm_hbm.at[b, c0 + r],
                         row_bufs[s].at[pl.ds(0, FH)], sem_in.at[s])
        pltpu.async_copy(vals_hbm.at[b, pl.ds((c0 + r) * NR, NR)],
                         vals_bufs[s], sem_in.at[s])

    start_in(0, 0)
    start_in(1, 1)
    start_in(2, 2)

    pltpu.sync_copy(ids_hbm.at[b], ids_v)

    def claim_body(g, _):
        k = ids_v[g >> 3, pl.ds((g & 7) * L, L)]
        nvec = g * L + iota
        _, last = plsc.scan_count(k)
        plsc.store_scatter(claim_v, [k], nvec, mask=last)
        return 0

    lax.fori_loop(0, NGROUPS, claim_body, 0)

    def win_body(g):
        k = ids_v[g >> 3, pl.ds((g & 7) * L, L)]
        nvec = g * L + iota
        w = plsc.load_gather(claim_v, [k])
        hw_v[pl.ds(g * L, L)] = jnp.where(w == nvec, k, (k & 127) | HW)

    plsc.parallel_loop(0, NGROUPS, unroll=2)(win_body)

    def wait_in(s):
        pltpu.make_async_copy(fm_hbm.at[b, c0],
                              row_bufs[s].at[pl.ds(0, FH)],
                              sem_in.at[s]).wait()
        pltpu.make_async_copy(vals_hbm.at[b, pl.ds(0, NR)], vals_bufs[s],
                              sem_in.at[s]).wait()

    def start_out(r, s):
        pltpu.async_copy(row_bufs[s].at[pl.ds(0, FH)],
                         out_hbm.at[b, c0 + r], sem_out.at[s])

    def wait_out(s):
        pltpu.make_async_copy(row_bufs[s].at[pl.ds(0, FH)],
                              out_hbm.at[b, c0], sem_out.at[s]).wait()

    def chunk_body(chunk):
        for s in range(NBUF):
            r = chunk * NBUF + s
            wait_in(s)
            s2 = (s + 3) % NBUF

            @pl.when(jnp.logical_and(r >= 1, r + 3 < C_PER_TILE))
            def _():
                wait_out(s2)

            @pl.when(r + 3 < C_PER_TILE)
            def _():
                start_in(r + 3, s2)

            def scat_body(g):
                hw = hw_v[pl.ds(g * L, L)]
                v = vals_bufs[s][g >> 3, pl.ds((g & 7) * L, L)]
                plsc.store_scatter(row_bufs[s], [hw >> 7, hw & 127], v)

            plsc.parallel_loop(0, NGROUPS, unroll=4)(scat_body)
            start_out(r, s)

    pl.loop(0, C_PER_TILE // NBUF)(chunk_body)

    for s in range(NBUF):
        wait_out(s)


@functools.partial(
    pl.kernel,
    out_type=jax.ShapeDtypeStruct((B, C, FH, FW), jnp.float32),
    mesh=plsc.VectorSubcoreMesh(core_axis_name="c", subcore_axis_name="s"),
    compiler_params=pltpu.CompilerParams(needs_layout_passes=False),
    scratch_types=[
        pltpu.VMEM((NR, FW), jnp.int32),
        pltpu.VMEM((HW,), jnp.int32),
        pltpu.VMEM((N,), jnp.int32),
        pltpu.VMEM((NR, FW), jnp.float32),
        pltpu.VMEM((NR, FW), jnp.float32),
        pltpu.VMEM((NR, FW), jnp.float32),
        pltpu.VMEM((NR, FW), jnp.float32),
        pltpu.VMEM((FH + PAD, FW), jnp.float32),
        pltpu.VMEM((FH + PAD, FW), jnp.float32),
        pltpu.VMEM((FH + PAD, FW), jnp.float32),
        pltpu.VMEM((FH + PAD, FW), jnp.float32),
        pltpu.SemaphoreType.DMA((NBUF,)),
        pltpu.SemaphoreType.DMA((NBUF,)),
    ],
)
def _sc_insert(fm_hbm, ids_hbm, vals_hbm, out_hbm,
               ids_v, claim_v, hw_v,
               vals_v0, vals_v1, vals_v2, vals_v3,
               row_v0, row_v1, row_v2, row_v3, sem_in, sem_out):
    _sc_body(fm_hbm, ids_hbm, vals_hbm, out_hbm,
             ids_v, claim_v, hw_v,
             [vals_v0, vals_v1, vals_v2, vals_v3],
             [row_v0, row_v1, row_v2, row_v3], sem_in, sem_out)


def kernel(feat_map, ins_ids, ins_feats):
    ids3 = ins_ids.astype(jnp.int32).reshape(B, NR, FW)
    vals3 = _transpose_feats(ins_feats.astype(jnp.float32))
    out4 = _sc_insert(feat_map, ids3, vals3)
    return out4
